# Initial kernel scaffold; baseline (speedup 1.0000x reference)
import jax
import jax.numpy as jnp
from jax.experimental import pallas as pl


def kernel(node_emb, basis, comp, root_w, conv_bias, ln_scale, ln_bias, Wq, Wk, Wv, Wo, bq, bk, bv, bo, score_w, score_b, edge_index, edge_type, drug_indices, pheno_indices, pheno_mask):
    raise NotImplementedError("write your pallas kernel here")


# speedup vs baseline: 12.0693x; 12.0693x over previous
import functools

import jax
import jax.numpy as jnp
from jax import lax
from jax.experimental import pallas as pl
from jax.experimental.pallas import tpu as pltpu
from jax.experimental.pallas import tpu_sc as plsc

N = 10000
R = 16
D = 128
NBASES = 10
NLAYERS = 2
NHEADS = 4
E = 320000
BATCH = 1024
P = 20
SEGS = N * R
YR = R + 1
HD = D // NHEADS

NC = 2
NS = 16
NW = NC * NS
EPW = E // NW
KE = 80
NBATCH = EPW // KE

AB = BATCH * (P + 1)
BPW = AB // NW
KG = 96
NGB = BPW // KG

NROWS_BLK = 1000
NBLK = N // NROWS_BLK
BB = 256
SEG_PER_SUB = SEGS // NS
N_PAD = 10240
ROWS_PER_SUB = N_PAD // NS
DRAIN = 128

_mesh = plsc.VectorSubcoreMesh(core_axis_name="c", subcore_axis_name="s")


def _wid():
    return lax.axis_index("s") * NC + lax.axis_index("c")


@functools.partial(
    pl.kernel,
    mesh=_mesh,
    out_type=[
        jax.ShapeDtypeStruct((E,), jnp.int32),
        jax.ShapeDtypeStruct((E,), jnp.int32),
        jax.ShapeDtypeStruct((NC * SEGS,), jnp.float32),
    ],
    scratch_types=[
        pltpu.VMEM((KE,), jnp.int32),
        pltpu.VMEM((KE,), jnp.int32),
        pltpu.VMEM((KE,), jnp.int32),
        pltpu.VMEM((KE,), jnp.int32),
        pltpu.VMEM((KE,), jnp.int32),
        pltpu.VMEM((KE,), jnp.float32),
        pltpu.VMEM((SEG_PER_SUB,), jnp.float32),
        pltpu.VMEM_SHARED((SEGS,), jnp.float32),
    ],
)
def _sc_prep(src_hbm, dst_hbm, rel_hbm, gidx_hbm, seg_hbm, cntp_hbm,
             src_v, dst_v, rel_v, g_v, s_v, ones_v, buf_v, cnt_sh):
    cid = lax.axis_index("c")
    sid = lax.axis_index("s")
    wid = _wid()

    def zbody(i, _):
        buf_v[pl.ds(i * 16, 16)] = jnp.zeros((16,), jnp.float32)
        ones_v[pl.ds((i % (KE // 16)) * 16, 16)] = jnp.ones((16,), jnp.float32)
        return 0

    lax.fori_loop(0, SEG_PER_SUB // 16, zbody, 0)
    pltpu.sync_copy(buf_v, cnt_sh.at[pl.ds(sid * SEG_PER_SUB, SEG_PER_SUB)])
    plsc.subcore_barrier()

    def batch_body(b, _):
        base = wid * EPW + b * KE
        pltpu.sync_copy(src_hbm.at[pl.ds(base, KE)], src_v)
        pltpu.sync_copy(dst_hbm.at[pl.ds(base, KE)], dst_v)
        pltpu.sync_copy(rel_hbm.at[pl.ds(base, KE)], rel_v)

        def vbody(i, _):
            sl = pl.ds(i * 16, 16)
            r = rel_v[sl]
            g_v[sl] = src_v[sl] * YR + r
            s_v[sl] = dst_v[sl] * R + r
            return 0

        lax.fori_loop(0, KE // 16, vbody, 0)
        pltpu.sync_copy(g_v, gidx_hbm.at[pl.ds(base, KE)])
        pltpu.sync_copy(s_v, seg_hbm.at[pl.ds(base, KE)])
        pltpu.sync_copy(ones_v, cnt_sh.at[s_v], add=True)
        return 0

    lax.fori_loop(0, NBATCH, batch_body, 0)
    plsc.subcore_barrier()

    pltpu.sync_copy(cnt_sh.at[pl.ds(sid * SEG_PER_SUB, SEG_PER_SUB)], buf_v)
    pltpu.sync_copy(buf_v, cntp_hbm.at[pl.ds(cid * SEGS + sid * SEG_PER_SUB,
                                             SEG_PER_SUB)])


@functools.partial(
    pl.kernel,
    mesh=_mesh,
    out_type=jax.ShapeDtypeStruct((NC, N_PAD, D), jnp.float32),
    scratch_types=[
        pltpu.VMEM((KE,), jnp.int32),
        pltpu.VMEM((KE,), jnp.int32),
        pltpu.VMEM((KE,), jnp.int32),
        pltpu.VMEM((KE,), jnp.float32),
        pltpu.VMEM((KE, D), jnp.float32),
        pltpu.VMEM((DRAIN, D), jnp.float32),
        pltpu.VMEM_SHARED((N_PAD, D), jnp.float32),
    ],
)
def _sc_scatter(y_hbm, gidx_hbm, seg_hbm, dst_hbm, inv_hbm, parts_hbm,
                gi_v, sg_v, ds_v, w_v, rows_v, dbuf, acc_sh):
    cid = lax.axis_index("c")
    sid = lax.axis_index("s")
    wid = _wid()

    def zb(i, _):
        for dd in range(D // 16):
            dbuf[i, pl.ds(dd * 16, 16)] = jnp.zeros((16,), jnp.float32)
        return 0

    lax.fori_loop(0, DRAIN, zb, 0)
    for i in range(ROWS_PER_SUB // DRAIN):
        pltpu.sync_copy(dbuf, acc_sh.at[pl.ds(sid * ROWS_PER_SUB + i * DRAIN, DRAIN)])
    plsc.subcore_barrier()

    def batch_body(b, _):
        base = wid * EPW + b * KE
        pltpu.sync_copy(gidx_hbm.at[pl.ds(base, KE)], gi_v)
        pltpu.sync_copy(seg_hbm.at[pl.ds(base, KE)], sg_v)
        pltpu.sync_copy(dst_hbm.at[pl.ds(base, KE)], ds_v)
        pltpu.sync_copy(inv_hbm.at[sg_v], w_v)
        pltpu.sync_copy(y_hbm.at[gi_v], rows_v)

        def rb(g, _):
            wg = w_v[pl.ds(g * 16, 16)]
            for l in range(16):
                j = g * 16 + l
                wspl = jnp.full((16,), wg[l], jnp.float32)
                for dd in range(D // 16):
                    sl = pl.ds(dd * 16, 16)
                    rows_v[j, sl] = rows_v[j, sl] * wspl
            return 0

        lax.fori_loop(0, KE // 16, rb, 0)
        pltpu.sync_copy(rows_v, acc_sh.at[ds_v], add=True)
        return 0

    lax.fori_loop(0, NBATCH, batch_body, 0)
    plsc.subcore_barrier()

    for i in range(ROWS_PER_SUB // DRAIN):
        sl = pl.ds(sid * ROWS_PER_SUB + i * DRAIN, DRAIN)
        pltpu.sync_copy(acc_sh.at[sl], dbuf)
        pltpu.sync_copy(dbuf, parts_hbm.at[cid, sl])


@functools.partial(
    pl.kernel,
    mesh=_mesh,
    out_type=jax.ShapeDtypeStruct((AB, D), jnp.float32),
    scratch_types=[
        pltpu.VMEM((KG,), jnp.int32),
        pltpu.VMEM((KG, D), jnp.float32),
    ],
)
def _sc_gather(x_hbm, idx_hbm, out_hbm, idx_v, rows_v):
    wid = _wid()

    def gb(i, _):
        base = wid * BPW + i * KG
        pltpu.sync_copy(idx_hbm.at[pl.ds(base, KG)], idx_v)
        pltpu.sync_copy(x_hbm.at[idx_v], rows_v)
        pltpu.sync_copy(rows_v, out_hbm.at[pl.ds(base, KG)])
        return 0

    lax.fori_loop(0, NGB, gb, 0)


def _cnt_body(cntp_ref, inv_ref):
    c = cntp_ref[0] + cntp_ref[1]
    inv_ref[...] = 1.0 / jnp.maximum(c, 1.0)


def _tc_invcnt(cntp):
    cntp3 = cntp.reshape(NC, SEGS // D, D)
    inv = pl.pallas_call(
        _cnt_body,
        out_shape=jax.ShapeDtypeStruct((SEGS // D, D), jnp.float32),
    )(cntp3)
    return inv.reshape(SEGS)


def _y_body(x_ref, basis_ref, comp_ref, root_ref, y_ref):
    x = x_ref[...]
    basis = basis_ref[...]
    comp = comp_ref[...]
    cols = []
    for r in range(R):
        w_r = jnp.tensordot(comp[r], basis, axes=[[0], [0]])
        cols.append(x @ w_r)
    cols.append(x @ root_ref[...])
    y_ref[...] = jnp.concatenate(cols, axis=1)


def _tc_y(x, basis_l, comp_l, root_l):
    return pl.pallas_call(
        _y_body,
        grid=(NBLK,),
        in_specs=[
            pl.BlockSpec((NROWS_BLK, D), lambda i: (i, 0)),
            pl.BlockSpec((NBASES, D, D), lambda i: (0, 0, 0)),
            pl.BlockSpec((R, NBASES), lambda i: (0, 0)),
            pl.BlockSpec((D, D), lambda i: (0, 0)),
        ],
        out_specs=pl.BlockSpec((NROWS_BLK, YR * D), lambda i: (i, 0)),
        out_shape=jax.ShapeDtypeStruct((N, YR * D), jnp.float32),
    )(x, basis_l, comp_l, root_l)


def _comb_body(parts_ref, yroot_ref, x_ref, cb_ref, ls_ref, lb_ref, o_ref):
    h = parts_ref[0] + parts_ref[1] + yroot_ref[...] + cb_ref[...]
    mu = jnp.mean(h, axis=-1, keepdims=True)
    var = jnp.mean((h - mu) ** 2, axis=-1, keepdims=True)
    hn = (h - mu) / jnp.sqrt(var + 1e-5) * ls_ref[...] + lb_ref[...]
    o_ref[...] = jnp.maximum(hn, 0.0) + x_ref[...]


def _tc_combine(parts, y, x, cb, ls, lb):
    return pl.pallas_call(
        _comb_body,
        grid=(NBLK,),
        in_specs=[
            pl.BlockSpec((NC, NROWS_BLK, D), lambda i: (0, i, 0)),
            pl.BlockSpec((NROWS_BLK, D), lambda i: (i, R)),
            pl.BlockSpec((NROWS_BLK, D), lambda i: (i, 0)),
            pl.BlockSpec((1, D), lambda i: (0, 0)),
            pl.BlockSpec((1, D), lambda i: (0, 0)),
            pl.BlockSpec((1, D), lambda i: (0, 0)),
        ],
        out_specs=pl.BlockSpec((NROWS_BLK, D), lambda i: (i, 0)),
        out_shape=jax.ShapeDtypeStruct((N, D), jnp.float32),
    )(parts, y, x, cb, ls, lb)


def _attn_body(drug_ref, ph_ref, mask_ref, wq_ref, wk_ref, wv_ref, wo_ref,
               bq_ref, bk_ref, bv_ref, bo_ref, sw_ref, sb_ref, out_ref):
    drug = drug_ref[...]
    ph = ph_ref[...]
    mask = mask_ref[...]
    q = drug @ wq_ref[...] + bq_ref[...]
    phf = ph.reshape(BB * P, D)
    k = (phf @ wk_ref[...] + bk_ref[...]).reshape(BB, P, D)
    v = (phf @ wv_ref[...] + bv_ref[...]).reshape(BB, P, D)
    scale = 1.0 / jnp.sqrt(jnp.float32(HD))
    ctxs = []
    for h in range(NHEADS):
        qh = q[:, h * HD:(h + 1) * HD]
        kh = k[:, :, h * HD:(h + 1) * HD]
        vh = v[:, :, h * HD:(h + 1) * HD]
        logit = jnp.sum(qh[:, None, :] * kh, axis=-1) * scale
        logit = jnp.where(mask > 0.0, jnp.float32(-1e9), logit)
        m = jnp.max(logit, axis=-1, keepdims=True)
        e = jnp.exp(logit - m)
        a = e / jnp.sum(e, axis=-1, keepdims=True)
        ctxs.append(jnp.sum(a[:, :, None] * vh, axis=1))
    ctx = jnp.concatenate(ctxs, axis=-1)
    out = ctx @ wo_ref[...] + bo_ref[...]
    score = jnp.sum(out * sw_ref[...], axis=-1) + sb_ref[0, 0]
    out_ref[...] = score[:, None]


def _tc_attn(drug, ph, maskf, Wq, Wk, Wv, Wo, bq, bk, bv, bo, sw, sb):
    full = lambda *shape: pl.BlockSpec(shape, lambda i: tuple(0 for _ in shape))
    return pl.pallas_call(
        _attn_body,
        grid=(BATCH // BB,),
        in_specs=[
            pl.BlockSpec((BB, D), lambda i: (i, 0)),
            pl.BlockSpec((BB, P, D), lambda i: (i, 0, 0)),
            pl.BlockSpec((BB, P), lambda i: (i, 0)),
            full(D, D), full(D, D), full(D, D), full(D, D),
            full(1, D), full(1, D), full(1, D), full(1, D),
            full(1, D), full(1, 1),
        ],
        out_specs=pl.BlockSpec((BB, 1), lambda i: (i, 0)),
        out_shape=jax.ShapeDtypeStruct((BATCH, 1), jnp.float32),
    )(drug, ph, maskf, Wq, Wk, Wv, Wo, bq, bk, bv, bo, sw, sb)


def kernel(node_emb, basis, comp, root_w, conv_bias, ln_scale, ln_bias,
           Wq, Wk, Wv, Wo, bq, bk, bv, bo, score_w, score_b,
           edge_index, edge_type, drug_indices, pheno_indices, pheno_mask):
    src = edge_index[0]
    dst = edge_index[1]
    gidx, seg, cntp = _sc_prep(src, dst, edge_type)
    inv = _tc_invcnt(cntp)

    x = node_emb
    for l in range(NLAYERS):
        y = _tc_y(x, basis[l], comp[l], root_w[l])
        y2 = y.reshape(N * YR, D)
        parts = _sc_scatter(y2, gidx, seg, dst, inv)
        x = _tc_combine(parts, y, x,
                        conv_bias[l].reshape(1, D),
                        ln_scale[l].reshape(1, D),
                        ln_bias[l].reshape(1, D))

    all_idx = jnp.concatenate([drug_indices, pheno_indices.reshape(-1)], axis=0)
    rows = _sc_gather(x, all_idx)
    drug = rows[:BATCH]
    ph = rows[BATCH:].reshape(BATCH, P, D)
    score = _tc_attn(drug, ph, pheno_mask.astype(jnp.float32),
                     Wq, Wk, Wv, Wo,
                     bq.reshape(1, D), bk.reshape(1, D), bv.reshape(1, D),
                     bo.reshape(1, D), score_w.reshape(1, D),
                     score_b.reshape(1, 1))
    return score.reshape(BATCH)


# speedup vs baseline: 19.2594x; 1.5957x over previous
import functools

import jax
import jax.numpy as jnp
from jax import lax
from jax.experimental import pallas as pl
from jax.experimental.pallas import tpu as pltpu
from jax.experimental.pallas import tpu_sc as plsc

N = 10000
R = 16
D = 128
NBASES = 10
NLAYERS = 2
NHEADS = 4
E = 320000
BATCH = 1024
P = 20
SEGS = N * R
YR = R + 1
HD = D // NHEADS

NC = 2
NS = 16
NW = NC * NS
EPW = E // NW
KE = 80
NBATCH = EPW // KE

AB = BATCH * (P + 1)
BPW = AB // NW
KG = 96
NGB = BPW // KG

NROWS_BLK = 1000
NBLK = N // NROWS_BLK
BB = 256
SEG_PER_SUB = SEGS // NS
N_PAD = 10240
ROWS_PER_SUB = N_PAD // NS
DRAIN = 32

_mesh = plsc.VectorSubcoreMesh(core_axis_name="c", subcore_axis_name="s")


def _wid():
    return lax.axis_index("s") * NC + lax.axis_index("c")


@functools.partial(
    pl.kernel,
    mesh=_mesh,
    out_type=[
        jax.ShapeDtypeStruct((E,), jnp.int32),
        jax.ShapeDtypeStruct((E,), jnp.int32),
        jax.ShapeDtypeStruct((NC * SEGS,), jnp.float32),
    ],
    scratch_types=[
        pltpu.VMEM((KE,), jnp.int32),
        pltpu.VMEM((KE,), jnp.int32),
        pltpu.VMEM((KE,), jnp.int32),
        pltpu.VMEM((KE,), jnp.int32),
        pltpu.VMEM((KE,), jnp.int32),
        pltpu.VMEM((KE,), jnp.float32),
        pltpu.VMEM((SEG_PER_SUB,), jnp.float32),
        pltpu.VMEM_SHARED((SEGS,), jnp.float32),
    ],
)
def _sc_prep(src_hbm, dst_hbm, rel_hbm, gidx_hbm, seg_hbm, cntp_hbm,
             src_v, dst_v, rel_v, g_v, s_v, ones_v, buf_v, cnt_sh):
    cid = lax.axis_index("c")
    sid = lax.axis_index("s")
    wid = _wid()

    def zbody(i, _):
        buf_v[pl.ds(i * 16, 16)] = jnp.zeros((16,), jnp.float32)
        ones_v[pl.ds((i % (KE // 16)) * 16, 16)] = jnp.ones((16,), jnp.float32)
        return 0

    lax.fori_loop(0, SEG_PER_SUB // 16, zbody, 0)
    pltpu.sync_copy(buf_v, cnt_sh.at[pl.ds(sid * SEG_PER_SUB, SEG_PER_SUB)])
    plsc.subcore_barrier()

    def batch_body(b, _):
        base = wid * EPW + b * KE
        pltpu.sync_copy(src_hbm.at[pl.ds(base, KE)], src_v)
        pltpu.sync_copy(dst_hbm.at[pl.ds(base, KE)], dst_v)
        pltpu.sync_copy(rel_hbm.at[pl.ds(base, KE)], rel_v)

        def vbody(i, _):
            sl = pl.ds(i * 16, 16)
            r = rel_v[sl]
            g_v[sl] = src_v[sl] * YR + r
            s_v[sl] = dst_v[sl] * R + r
            return 0

        lax.fori_loop(0, KE // 16, vbody, 0)
        pltpu.sync_copy(g_v, gidx_hbm.at[pl.ds(base, KE)])
        pltpu.sync_copy(s_v, seg_hbm.at[pl.ds(base, KE)])
        pltpu.sync_copy(ones_v, cnt_sh.at[s_v], add=True)
        return 0

    lax.fori_loop(0, NBATCH, batch_body, 0)
    plsc.subcore_barrier()

    pltpu.sync_copy(cnt_sh.at[pl.ds(sid * SEG_PER_SUB, SEG_PER_SUB)], buf_v)
    pltpu.sync_copy(buf_v, cntp_hbm.at[pl.ds(cid * SEGS + sid * SEG_PER_SUB,
                                             SEG_PER_SUB)])


KW = 25


@functools.partial(
    pl.kernel,
    mesh=_mesh,
    out_type=jax.ShapeDtypeStruct((E,), jnp.float32),
    scratch_types=[
        pltpu.VMEM((EPW,), jnp.int32),
        pltpu.VMEM((NBATCH, KE), jnp.float32),
        pltpu.SemaphoreType.DMA,
        pltpu.SemaphoreType.DMA,
    ],
)
def _sc_wprep(seg_hbm, inv_hbm, w_hbm, seg_v, w2_v, gsem, ssem):
    wid = _wid()
    ebase = wid * EPW
    pltpu.sync_copy(seg_hbm.at[pl.ds(ebase, EPW)], seg_v)

    def chunk(c, _):
        def fire(j, _):
            jj = c * KW + j
            pltpu.async_copy(inv_hbm.at[seg_v.at[pl.ds(jj * KE, KE)]],
                             w2_v.at[jj], gsem)
            return 0

        def gwait(j, _):
            jj = c * KW + j
            pltpu.make_async_copy(inv_hbm.at[seg_v.at[pl.ds(jj * KE, KE)]],
                                  w2_v.at[jj], gsem).wait()
            return 0

        def sfire(j, _):
            jj = c * KW + j
            pltpu.async_copy(w2_v.at[jj], w_hbm.at[pl.ds(ebase + jj * KE, KE)],
                             ssem)
            return 0

        def swait(j, _):
            jj = c * KW + j
            pltpu.make_async_copy(w2_v.at[jj],
                                  w_hbm.at[pl.ds(ebase + jj * KE, KE)],
                                  ssem).wait()
            return 0

        lax.fori_loop(0, KW, fire, 0)
        lax.fori_loop(0, KW, gwait, 0)
        lax.fori_loop(0, KW, sfire, 0)
        lax.fori_loop(0, KW, swait, 0)
        return 0

    lax.fori_loop(0, NBATCH // KW, chunk, 0)


@functools.partial(
    pl.kernel,
    mesh=_mesh,
    out_type=jax.ShapeDtypeStruct((NC, N_PAD, D), jnp.float32),
    scratch_types=[
        pltpu.VMEM((EPW,), jnp.int32),
        pltpu.VMEM((EPW,), jnp.float32),
        pltpu.VMEM((2, KE), jnp.int32),
        pltpu.VMEM((2, KE, D), jnp.float32),
        pltpu.VMEM((DRAIN, D), jnp.float32),
        pltpu.VMEM_SHARED((N_PAD, D), jnp.float32),
        pltpu.SemaphoreType.DMA((2,)),
        pltpu.SemaphoreType.DMA((2,)),
        pltpu.SemaphoreType.DMA((2,)),
    ],
)
def _sc_scatter(y_hbm, gidx_hbm, dst_hbm, w_hbm, parts_hbm,
                gidx_v, w_v, dstr_v, rows_v, dbuf, acc_sh,
                dsems, rsems, ssems):
    cid = lax.axis_index("c")
    sid = lax.axis_index("s")
    wid = _wid()
    ebase = wid * EPW

    pltpu.sync_copy(gidx_hbm.at[pl.ds(ebase, EPW)], gidx_v)
    pltpu.sync_copy(w_hbm.at[pl.ds(ebase, EPW)], w_v)

    def zb(i, _):
        for dd in range(D // 16):
            dbuf[i, pl.ds(dd * 16, 16)] = jnp.zeros((16,), jnp.float32)
        return 0

    lax.fori_loop(0, DRAIN, zb, 0)
    for i in range(ROWS_PER_SUB // DRAIN):
        pltpu.sync_copy(dbuf, acc_sh.at[pl.ds(sid * ROWS_PER_SUB + i * DRAIN, DRAIN)])
    plsc.subcore_barrier()

    def fire_batch(j, p):
        pltpu.async_copy(dst_hbm.at[pl.ds(ebase + j * KE, KE)], dstr_v.at[p],
                         dsems.at[p])
        pltpu.async_copy(y_hbm.at[gidx_v.at[pl.ds(j * KE, KE)]], rows_v.at[p],
                         rsems.at[p])

    def wait_batch(j, p):
        pltpu.make_async_copy(dst_hbm.at[pl.ds(ebase + j * KE, KE)],
                              dstr_v.at[p], dsems.at[p]).wait()
        pltpu.make_async_copy(y_hbm.at[gidx_v.at[pl.ds(j * KE, KE)]],
                              rows_v.at[p], rsems.at[p]).wait()

    def wait_scat(p):
        pltpu.make_async_copy(rows_v.at[p], acc_sh.at[dstr_v.at[p]],
                              ssems.at[p]).wait()

    def process(j, p):
        wait_batch(j, p)

        def rb(g2, _):
            wg = w_v[pl.ds(j * KE + g2 * 16, 16)]
            for l in range(16):
                wspl = jnp.full((16,), wg[l], jnp.float32)
                for dd in range(D // 16):
                    sl = pl.ds(dd * 16, 16)
                    rows_v[p, g2 * 16 + l, sl] = rows_v[p, g2 * 16 + l, sl] * wspl
            return 0

        lax.fori_loop(0, KE // 16, rb, 0)
        pltpu.async_copy(rows_v.at[p], acc_sh.at[dstr_v.at[p]], ssems.at[p],
                         add=True)

    fire_batch(0, 0)
    fire_batch(1, 1)

    def group_body(g, _):
        for p in range(2):
            j = 2 * g + p
            process(j, p)

            @pl.when(j + 2 < NBATCH)
            def _():
                wait_scat(p)
                fire_batch(j + 2, p)

        return 0

    lax.fori_loop(0, (NBATCH - 1) // 2, group_body, 0)
    process(NBATCH - 1, 0)
    wait_scat(0)
    wait_scat(1)
    plsc.subcore_barrier()

    for i in range(ROWS_PER_SUB // DRAIN):
        sl = pl.ds(sid * ROWS_PER_SUB + i * DRAIN, DRAIN)
        pltpu.sync_copy(acc_sh.at[sl], dbuf)
        pltpu.sync_copy(dbuf, parts_hbm.at[cid, sl])


@functools.partial(
    pl.kernel,
    mesh=_mesh,
    out_type=jax.ShapeDtypeStruct((AB, D), jnp.float32),
    scratch_types=[
        pltpu.VMEM((KG,), jnp.int32),
        pltpu.VMEM((KG, D), jnp.float32),
    ],
)
def _sc_gather(x_hbm, idx_hbm, out_hbm, idx_v, rows_v):
    wid = _wid()

    def gb(i, _):
        base = wid * BPW + i * KG
        pltpu.sync_copy(idx_hbm.at[pl.ds(base, KG)], idx_v)
        pltpu.sync_copy(x_hbm.at[idx_v], rows_v)
        pltpu.sync_copy(rows_v, out_hbm.at[pl.ds(base, KG)])
        return 0

    lax.fori_loop(0, NGB, gb, 0)


def _cnt_body(cntp_ref, inv_ref):
    c = cntp_ref[0] + cntp_ref[1]
    inv_ref[...] = 1.0 / jnp.maximum(c, 1.0)


def _tc_invcnt(cntp):
    cntp3 = cntp.reshape(NC, SEGS // D, D)
    inv = pl.pallas_call(
        _cnt_body,
        out_shape=jax.ShapeDtypeStruct((SEGS // D, D), jnp.float32),
    )(cntp3)
    return inv.reshape(SEGS)


def _y_body(x_ref, basis_ref, comp_ref, root_ref, y_ref):
    x = x_ref[...]
    comp = comp_ref[...]
    xbs = [x @ basis_ref[b] for b in range(NBASES)]
    cols = []
    for r in range(R):
        acc = comp[r, 0] * xbs[0]
        for b in range(1, NBASES):
            acc = acc + comp[r, b] * xbs[b]
        cols.append(acc)
    cols.append(x @ root_ref[...])
    y_ref[...] = jnp.concatenate(cols, axis=1)


def _tc_y(x, basis_l, comp_l, root_l):
    return pl.pallas_call(
        _y_body,
        grid=(NBLK,),
        in_specs=[
            pl.BlockSpec((NROWS_BLK, D), lambda i: (i, 0)),
            pl.BlockSpec((NBASES, D, D), lambda i: (0, 0, 0)),
            pl.BlockSpec((R, NBASES), lambda i: (0, 0)),
            pl.BlockSpec((D, D), lambda i: (0, 0)),
        ],
        out_specs=pl.BlockSpec((NROWS_BLK, YR * D), lambda i: (i, 0)),
        out_shape=jax.ShapeDtypeStruct((N, YR * D), jnp.float32),
    )(x, basis_l, comp_l, root_l)


def _comb_body(parts_ref, yroot_ref, x_ref, cb_ref, ls_ref, lb_ref, o_ref):
    h = parts_ref[0] + parts_ref[1] + yroot_ref[...] + cb_ref[...]
    mu = jnp.mean(h, axis=-1, keepdims=True)
    var = jnp.mean((h - mu) ** 2, axis=-1, keepdims=True)
    hn = (h - mu) / jnp.sqrt(var + 1e-5) * ls_ref[...] + lb_ref[...]
    o_ref[...] = jnp.maximum(hn, 0.0) + x_ref[...]


def _tc_combine(parts, y, x, cb, ls, lb):
    return pl.pallas_call(
        _comb_body,
        grid=(NBLK,),
        in_specs=[
            pl.BlockSpec((NC, NROWS_BLK, D), lambda i: (0, i, 0)),
            pl.BlockSpec((NROWS_BLK, D), lambda i: (i, R)),
            pl.BlockSpec((NROWS_BLK, D), lambda i: (i, 0)),
            pl.BlockSpec((1, D), lambda i: (0, 0)),
            pl.BlockSpec((1, D), lambda i: (0, 0)),
            pl.BlockSpec((1, D), lambda i: (0, 0)),
        ],
        out_specs=pl.BlockSpec((NROWS_BLK, D), lambda i: (i, 0)),
        out_shape=jax.ShapeDtypeStruct((N, D), jnp.float32),
    )(parts, y, x, cb, ls, lb)


def _attn_body(drug_ref, ph_ref, mask_ref, wq_ref, wk_ref, wv_ref, wo_ref,
               bq_ref, bk_ref, bv_ref, bo_ref, sw_ref, sb_ref, out_ref):
    drug = drug_ref[...]
    ph = ph_ref[...]
    mask = mask_ref[...]
    q = drug @ wq_ref[...] + bq_ref[...]
    phf = ph.reshape(BB * P, D)
    k = (phf @ wk_ref[...] + bk_ref[...]).reshape(BB, P, D)
    v = (phf @ wv_ref[...] + bv_ref[...]).reshape(BB, P, D)
    scale = 1.0 / jnp.sqrt(jnp.float32(HD))
    ctxs = []
    for h in range(NHEADS):
        qh = q[:, h * HD:(h + 1) * HD]
        kh = k[:, :, h * HD:(h + 1) * HD]
        vh = v[:, :, h * HD:(h + 1) * HD]
        logit = jnp.sum(qh[:, None, :] * kh, axis=-1) * scale
        logit = jnp.where(mask > 0.0, jnp.float32(-1e9), logit)
        m = jnp.max(logit, axis=-1, keepdims=True)
        e = jnp.exp(logit - m)
        a = e / jnp.sum(e, axis=-1, keepdims=True)
        ctxs.append(jnp.sum(a[:, :, None] * vh, axis=1))
    ctx = jnp.concatenate(ctxs, axis=-1)
    out = ctx @ wo_ref[...] + bo_ref[...]
    score = jnp.sum(out * sw_ref[...], axis=-1) + sb_ref[0, 0]
    out_ref[...] = score[:, None]


def _tc_attn(drug, ph, maskf, Wq, Wk, Wv, Wo, bq, bk, bv, bo, sw, sb):
    full = lambda *shape: pl.BlockSpec(shape, lambda i: tuple(0 for _ in shape))
    return pl.pallas_call(
        _attn_body,
        grid=(BATCH // BB,),
        in_specs=[
            pl.BlockSpec((BB, D), lambda i: (i, 0)),
            pl.BlockSpec((BB, P, D), lambda i: (i, 0, 0)),
            pl.BlockSpec((BB, P), lambda i: (i, 0)),
            full(D, D), full(D, D), full(D, D), full(D, D),
            full(1, D), full(1, D), full(1, D), full(1, D),
            full(1, D), full(1, 1),
        ],
        out_specs=pl.BlockSpec((BB, 1), lambda i: (i, 0)),
        out_shape=jax.ShapeDtypeStruct((BATCH, 1), jnp.float32),
    )(drug, ph, maskf, Wq, Wk, Wv, Wo, bq, bk, bv, bo, sw, sb)


def kernel(node_emb, basis, comp, root_w, conv_bias, ln_scale, ln_bias,
           Wq, Wk, Wv, Wo, bq, bk, bv, bo, score_w, score_b,
           edge_index, edge_type, drug_indices, pheno_indices, pheno_mask):
    src = edge_index[0]
    dst = edge_index[1]
    gidx, seg, cntp = _sc_prep(src, dst, edge_type)
    inv = _tc_invcnt(cntp)
    w = _sc_wprep(seg, inv)

    x = node_emb
    for l in range(NLAYERS):
        y = _tc_y(x, basis[l], comp[l], root_w[l])
        y2 = y.reshape(N * YR, D)
        parts = _sc_scatter(y2, gidx, dst, w)
        x = _tc_combine(parts, y, x,
                        conv_bias[l].reshape(1, D),
                        ln_scale[l].reshape(1, D),
                        ln_bias[l].reshape(1, D))

    all_idx = jnp.concatenate([drug_indices, pheno_indices.reshape(-1)], axis=0)
    rows = _sc_gather(x, all_idx)
    drug = rows[:BATCH]
    ph = rows[BATCH:].reshape(BATCH, P, D)
    score = _tc_attn(drug, ph, pheno_mask.astype(jnp.float32),
                     Wq, Wk, Wv, Wo,
                     bq.reshape(1, D), bk.reshape(1, D), bv.reshape(1, D),
                     bo.reshape(1, D), score_w.reshape(1, D),
                     score_b.reshape(1, 1))
    return score.reshape(BATCH)


# speedup vs baseline: 23.6548x; 1.2282x over previous
import functools

import jax
import jax.numpy as jnp
from jax import lax
from jax.experimental import pallas as pl
from jax.experimental.pallas import tpu as pltpu
from jax.experimental.pallas import tpu_sc as plsc

N = 10000
R = 16
D = 128
NBASES = 10
NLAYERS = 2
NHEADS = 4
E = 320000
BATCH = 1024
P = 20
SEGS = N * R
YR = R + 1
HD = D // NHEADS

NC = 2
NS = 16
NW = NC * NS
EPW = E // NW
KE = 80
NBATCH = EPW // KE

AB = BATCH * (P + 1)
BPW = AB // NW
KG = 96
NGB = BPW // KG

NROWS_BLK = 1000
NBLK = N // NROWS_BLK
BB = 256
SEG_PER_SUB = SEGS // NS
N_PAD = 10240
ROWS_PER_SUB = N_PAD // NS
DRAIN = 32
KW = 25

_mesh = plsc.VectorSubcoreMesh(core_axis_name="c", subcore_axis_name="s")


def _wid():
    return lax.axis_index("s") * NC + lax.axis_index("c")


@functools.partial(
    pl.kernel,
    mesh=_mesh,
    out_type=[
        jax.ShapeDtypeStruct((E,), jnp.int32),
        jax.ShapeDtypeStruct((E,), jnp.int32),
        jax.ShapeDtypeStruct((NC * SEGS,), jnp.float32),
    ],
    scratch_types=[
        pltpu.VMEM((EPW,), jnp.int32),
        pltpu.VMEM((EPW,), jnp.int32),
        pltpu.VMEM((EPW,), jnp.int32),
        pltpu.VMEM((EPW,), jnp.int32),
        pltpu.VMEM((EPW,), jnp.int32),
        pltpu.VMEM((NBATCH, KE), jnp.int32),
        pltpu.VMEM((KE,), jnp.float32),
        pltpu.VMEM((SEG_PER_SUB,), jnp.float32),
        pltpu.VMEM_SHARED((SEGS,), jnp.float32),
        pltpu.SemaphoreType.DMA,
    ],
)
def _sc_prep(src_hbm, dst_hbm, rel_hbm, gidx_hbm, seg_hbm, cntp_hbm,
             src_v, dst_v, rel_v, g_v, s_v, s2_v, ones_v, buf_v, cnt_sh, hsem):
    cid = lax.axis_index("c")
    sid = lax.axis_index("s")
    wid = _wid()
    ebase = wid * EPW

    def zbody(i, _):
        buf_v[pl.ds(i * 16, 16)] = jnp.zeros((16,), jnp.float32)
        return 0

    lax.fori_loop(0, SEG_PER_SUB // 16, zbody, 0)
    for i in range(KE // 16):
        ones_v[pl.ds(i * 16, 16)] = jnp.ones((16,), jnp.float32)
    pltpu.sync_copy(buf_v, cnt_sh.at[pl.ds(sid * SEG_PER_SUB, SEG_PER_SUB)])

    pltpu.sync_copy(src_hbm.at[pl.ds(ebase, EPW)], src_v)
    pltpu.sync_copy(dst_hbm.at[pl.ds(ebase, EPW)], dst_v)
    pltpu.sync_copy(rel_hbm.at[pl.ds(ebase, EPW)], rel_v)

    def vbody(row, _):
        for c in range(KE // 16):
            sl = pl.ds(row * KE + c * 16, 16)
            r = rel_v[sl]
            g_v[sl] = src_v[sl] * YR + r
            seg16 = dst_v[sl] * R + r
            s_v[sl] = seg16
            s2_v[row, pl.ds(c * 16, 16)] = seg16
        return 0

    lax.fori_loop(0, NBATCH, vbody, 0)
    pltpu.sync_copy(g_v, gidx_hbm.at[pl.ds(ebase, EPW)])
    pltpu.sync_copy(s_v, seg_hbm.at[pl.ds(ebase, EPW)])
    plsc.subcore_barrier()

    def hchunk(c, _):
        def hfire(j, _):
            pltpu.async_copy(ones_v, cnt_sh.at[s2_v.at[c * KW + j]], hsem,
                             add=True)
            return 0

        def hwait(j, _):
            pltpu.make_async_copy(ones_v, cnt_sh.at[s2_v.at[c * KW + j]],
                                  hsem).wait()
            return 0

        lax.fori_loop(0, KW, hfire, 0)
        lax.fori_loop(0, KW, hwait, 0)
        return 0

    lax.fori_loop(0, NBATCH // KW, hchunk, 0)
    plsc.subcore_barrier()

    pltpu.sync_copy(cnt_sh.at[pl.ds(sid * SEG_PER_SUB, SEG_PER_SUB)], buf_v)
    pltpu.sync_copy(buf_v, cntp_hbm.at[pl.ds(cid * SEGS + sid * SEG_PER_SUB,
                                             SEG_PER_SUB)])


@functools.partial(
    pl.kernel,
    mesh=_mesh,
    out_type=jax.ShapeDtypeStruct((E,), jnp.float32),
    scratch_types=[
        pltpu.VMEM((EPW,), jnp.int32),
        pltpu.VMEM((NBATCH, KE), jnp.float32),
        pltpu.SemaphoreType.DMA,
        pltpu.SemaphoreType.DMA,
    ],
)
def _sc_wprep(seg_hbm, inv_hbm, w_hbm, seg_v, w2_v, gsem, ssem):
    wid = _wid()
    ebase = wid * EPW
    pltpu.sync_copy(seg_hbm.at[pl.ds(ebase, EPW)], seg_v)

    def chunk(c, _):
        def fire(j, _):
            jj = c * KW + j
            pltpu.async_copy(inv_hbm.at[seg_v.at[pl.ds(jj * KE, KE)]],
                             w2_v.at[jj], gsem)
            return 0

        def gwait(j, _):
            jj = c * KW + j
            pltpu.make_async_copy(inv_hbm.at[seg_v.at[pl.ds(jj * KE, KE)]],
                                  w2_v.at[jj], gsem).wait()
            return 0

        def sfire(j, _):
            jj = c * KW + j
            pltpu.async_copy(w2_v.at[jj], w_hbm.at[pl.ds(ebase + jj * KE, KE)],
                             ssem)
            return 0

        def swait(j, _):
            jj = c * KW + j
            pltpu.make_async_copy(w2_v.at[jj],
                                  w_hbm.at[pl.ds(ebase + jj * KE, KE)],
                                  ssem).wait()
            return 0

        lax.fori_loop(0, KW, fire, 0)
        lax.fori_loop(0, KW, gwait, 0)
        lax.fori_loop(0, KW, sfire, 0)
        lax.fori_loop(0, KW, swait, 0)
        return 0

    lax.fori_loop(0, NBATCH // KW, chunk, 0)


@functools.partial(
    pl.kernel,
    mesh=_mesh,
    out_type=jax.ShapeDtypeStruct((NC, N_PAD, D), jnp.float32),
    scratch_types=[
        pltpu.VMEM((EPW,), jnp.int32),
        pltpu.VMEM((EPW,), jnp.float32),
        pltpu.VMEM((2, KE), jnp.int32),
        pltpu.VMEM((2, KE, D), jnp.float32),
        pltpu.VMEM((DRAIN, D), jnp.float32),
        pltpu.VMEM_SHARED((N_PAD, D), jnp.float32),
        pltpu.SemaphoreType.DMA((2,)),
        pltpu.SemaphoreType.DMA((2,)),
        pltpu.SemaphoreType.DMA((2,)),
    ],
)
def _sc_scatter(y_hbm, gidx_hbm, dst_hbm, w_hbm, parts_hbm,
                gidx_v, w_v, dstr_v, rows_v, dbuf, acc_sh,
                dsems, rsems, ssems):
    cid = lax.axis_index("c")
    sid = lax.axis_index("s")
    wid = _wid()
    ebase = wid * EPW

    pltpu.sync_copy(gidx_hbm.at[pl.ds(ebase, EPW)], gidx_v)
    pltpu.sync_copy(w_hbm.at[pl.ds(ebase, EPW)], w_v)

    def zb(i, _):
        for dd in range(D // 16):
            dbuf[i, pl.ds(dd * 16, 16)] = jnp.zeros((16,), jnp.float32)
        return 0

    lax.fori_loop(0, DRAIN, zb, 0)
    for i in range(ROWS_PER_SUB // DRAIN):
        pltpu.sync_copy(dbuf, acc_sh.at[pl.ds(sid * ROWS_PER_SUB + i * DRAIN, DRAIN)])
    plsc.subcore_barrier()

    def fire_batch(j, p):
        pltpu.async_copy(dst_hbm.at[pl.ds(ebase + j * KE, KE)], dstr_v.at[p],
                         dsems.at[p])
        pltpu.async_copy(y_hbm.at[gidx_v.at[pl.ds(j * KE, KE)]], rows_v.at[p],
                         rsems.at[p])

    def wait_batch(j, p):
        pltpu.make_async_copy(dst_hbm.at[pl.ds(ebase + j * KE, KE)],
                              dstr_v.at[p], dsems.at[p]).wait()
        pltpu.make_async_copy(y_hbm.at[gidx_v.at[pl.ds(j * KE, KE)]],
                              rows_v.at[p], rsems.at[p]).wait()

    def wait_scat(p):
        pltpu.make_async_copy(rows_v.at[p], acc_sh.at[dstr_v.at[p]],
                              ssems.at[p]).wait()

    def process(j, p):
        wait_batch(j, p)

        def rb(g2, _):
            wg = w_v[pl.ds(j * KE + g2 * 16, 16)]
            for l in range(16):
                wspl = jnp.full((16,), wg[l], jnp.float32)
                for dd in range(D // 16):
                    sl = pl.ds(dd * 16, 16)
                    rows_v[p, g2 * 16 + l, sl] = rows_v[p, g2 * 16 + l, sl] * wspl
            return 0

        lax.fori_loop(0, KE // 16, rb, 0)
        pltpu.async_copy(rows_v.at[p], acc_sh.at[dstr_v.at[p]], ssems.at[p],
                         add=True)

    fire_batch(0, 0)
    fire_batch(1, 1)

    def group_body(g, _):
        for p in range(2):
            j = 2 * g + p
            process(j, p)

            @pl.when(j + 2 < NBATCH)
            def _():
                wait_scat(p)
                fire_batch(j + 2, p)

        return 0

    lax.fori_loop(0, (NBATCH - 1) // 2, group_body, 0)
    process(NBATCH - 1, 0)
    wait_scat(0)
    wait_scat(1)
    plsc.subcore_barrier()

    for i in range(ROWS_PER_SUB // DRAIN):
        sl = pl.ds(sid * ROWS_PER_SUB + i * DRAIN, DRAIN)
        pltpu.sync_copy(acc_sh.at[sl], dbuf)
        pltpu.sync_copy(dbuf, parts_hbm.at[cid, sl])


@functools.partial(
    pl.kernel,
    mesh=_mesh,
    out_type=jax.ShapeDtypeStruct((AB, D), jnp.float32),
    scratch_types=[
        pltpu.VMEM((KG,), jnp.int32),
        pltpu.VMEM((KG, D), jnp.float32),
    ],
)
def _sc_gather(x_hbm, idx_hbm, out_hbm, idx_v, rows_v):
    wid = _wid()

    def gb(i, _):
        base = wid * BPW + i * KG
        pltpu.sync_copy(idx_hbm.at[pl.ds(base, KG)], idx_v)
        pltpu.sync_copy(x_hbm.at[idx_v], rows_v)
        pltpu.sync_copy(rows_v, out_hbm.at[pl.ds(base, KG)])
        return 0

    lax.fori_loop(0, NGB, gb, 0)


def _cnt_body(cntp_ref, inv_ref):
    c = cntp_ref[0] + cntp_ref[1]
    inv_ref[...] = 1.0 / jnp.maximum(c, 1.0)


def _tc_invcnt(cntp):
    cntp3 = cntp.reshape(NC, SEGS // D, D)
    inv = pl.pallas_call(
        _cnt_body,
        out_shape=jax.ShapeDtypeStruct((SEGS // D, D), jnp.float32),
    )(cntp3)
    return inv.reshape(SEGS)


def _y_body(x_ref, basis_ref, comp_ref, root_ref, y_ref):
    x = x_ref[...]
    comp = comp_ref[...]
    xbs = [x @ basis_ref[b] for b in range(NBASES)]
    cols = []
    for r in range(R):
        acc = comp[r, 0] * xbs[0]
        for b in range(1, NBASES):
            acc = acc + comp[r, b] * xbs[b]
        cols.append(acc)
    cols.append(x @ root_ref[...])
    y_ref[...] = jnp.concatenate(cols, axis=1)


def _tc_y(x, basis_l, comp_l, root_l):
    return pl.pallas_call(
        _y_body,
        grid=(NBLK,),
        in_specs=[
            pl.BlockSpec((NROWS_BLK, D), lambda i: (i, 0)),
            pl.BlockSpec((NBASES, D, D), lambda i: (0, 0, 0)),
            pl.BlockSpec((R, NBASES), lambda i: (0, 0)),
            pl.BlockSpec((D, D), lambda i: (0, 0)),
        ],
        out_specs=pl.BlockSpec((NROWS_BLK, YR * D), lambda i: (i, 0)),
        out_shape=jax.ShapeDtypeStruct((N, YR * D), jnp.float32),
    )(x, basis_l, comp_l, root_l)


def _comb_body(parts_ref, yroot_ref, x_ref, cb_ref, ls_ref, lb_ref, o_ref):
    h = parts_ref[0] + parts_ref[1] + yroot_ref[...] + cb_ref[...]
    mu = jnp.mean(h, axis=-1, keepdims=True)
    var = jnp.mean((h - mu) ** 2, axis=-1, keepdims=True)
    hn = (h - mu) / jnp.sqrt(var + 1e-5) * ls_ref[...] + lb_ref[...]
    o_ref[...] = jnp.maximum(hn, 0.0) + x_ref[...]


def _tc_combine(parts, y, x, cb, ls, lb):
    return pl.pallas_call(
        _comb_body,
        grid=(NBLK,),
        in_specs=[
            pl.BlockSpec((NC, NROWS_BLK, D), lambda i: (0, i, 0)),
            pl.BlockSpec((NROWS_BLK, D), lambda i: (i, R)),
            pl.BlockSpec((NROWS_BLK, D), lambda i: (i, 0)),
            pl.BlockSpec((1, D), lambda i: (0, 0)),
            pl.BlockSpec((1, D), lambda i: (0, 0)),
            pl.BlockSpec((1, D), lambda i: (0, 0)),
        ],
        out_specs=pl.BlockSpec((NROWS_BLK, D), lambda i: (i, 0)),
        out_shape=jax.ShapeDtypeStruct((N, D), jnp.float32),
    )(parts, y, x, cb, ls, lb)


def _attn_body(drug_ref, ph_ref, mask_ref, wq_ref, wk_ref, wv_ref, wo_ref,
               bq_ref, bk_ref, bv_ref, bo_ref, sw_ref, sb_ref, out_ref):
    drug = drug_ref[...]
    ph = ph_ref[...]
    mask = mask_ref[...]
    q = drug @ wq_ref[...] + bq_ref[...]
    phf = ph.reshape(BB * P, D)
    k = (phf @ wk_ref[...] + bk_ref[...]).reshape(BB, P, D)
    v = (phf @ wv_ref[...] + bv_ref[...]).reshape(BB, P, D)
    scale = 1.0 / jnp.sqrt(jnp.float32(HD))
    ctxs = []
    for h in range(NHEADS):
        qh = q[:, h * HD:(h + 1) * HD]
        kh = k[:, :, h * HD:(h + 1) * HD]
        vh = v[:, :, h * HD:(h + 1) * HD]
        logit = jnp.sum(qh[:, None, :] * kh, axis=-1) * scale
        logit = jnp.where(mask > 0.0, jnp.float32(-1e9), logit)
        m = jnp.max(logit, axis=-1, keepdims=True)
        e = jnp.exp(logit - m)
        a = e / jnp.sum(e, axis=-1, keepdims=True)
        ctxs.append(jnp.sum(a[:, :, None] * vh, axis=1))
    ctx = jnp.concatenate(ctxs, axis=-1)
    out = ctx @ wo_ref[...] + bo_ref[...]
    score = jnp.sum(out * sw_ref[...], axis=-1) + sb_ref[0, 0]
    out_ref[...] = score[:, None]


def _tc_attn(drug, ph, maskf, Wq, Wk, Wv, Wo, bq, bk, bv, bo, sw, sb):
    full = lambda *shape: pl.BlockSpec(shape, lambda i: tuple(0 for _ in shape))
    return pl.pallas_call(
        _attn_body,
        grid=(BATCH // BB,),
        in_specs=[
            pl.BlockSpec((BB, D), lambda i: (i, 0)),
            pl.BlockSpec((BB, P, D), lambda i: (i, 0, 0)),
            pl.BlockSpec((BB, P), lambda i: (i, 0)),
            full(D, D), full(D, D), full(D, D), full(D, D),
            full(1, D), full(1, D), full(1, D), full(1, D),
            full(1, D), full(1, 1),
        ],
        out_specs=pl.BlockSpec((BB, 1), lambda i: (i, 0)),
        out_shape=jax.ShapeDtypeStruct((BATCH, 1), jnp.float32),
    )(drug, ph, maskf, Wq, Wk, Wv, Wo, bq, bk, bv, bo, sw, sb)


def kernel(node_emb, basis, comp, root_w, conv_bias, ln_scale, ln_bias,
           Wq, Wk, Wv, Wo, bq, bk, bv, bo, score_w, score_b,
           edge_index, edge_type, drug_indices, pheno_indices, pheno_mask):
    src = edge_index[0]
    dst = edge_index[1]
    gidx, seg, cntp = _sc_prep(src, dst, edge_type)
    inv = _tc_invcnt(cntp)
    w = _sc_wprep(seg, inv)

    x = node_emb
    for l in range(NLAYERS):
        y = _tc_y(x, basis[l], comp[l], root_w[l])
        y2 = y.reshape(N * YR, D)
        parts = _sc_scatter(y2, gidx, dst, w)
        x = _tc_combine(parts, y, x,
                        conv_bias[l].reshape(1, D),
                        ln_scale[l].reshape(1, D),
                        ln_bias[l].reshape(1, D))

    all_idx = jnp.concatenate([drug_indices, pheno_indices.reshape(-1)], axis=0)
    rows = _sc_gather(x, all_idx)
    drug = rows[:BATCH]
    ph = rows[BATCH:].reshape(BATCH, P, D)
    score = _tc_attn(drug, ph, pheno_mask.astype(jnp.float32),
                     Wq, Wk, Wv, Wo,
                     bq.reshape(1, D), bk.reshape(1, D), bv.reshape(1, D),
                     bo.reshape(1, D), score_w.reshape(1, D),
                     score_b.reshape(1, 1))
    return score.reshape(BATCH)


# speedup vs baseline: 23.8347x; 1.0076x over previous
import functools

import jax
import jax.numpy as jnp
from jax import lax
from jax.experimental import pallas as pl
from jax.experimental.pallas import tpu as pltpu
from jax.experimental.pallas import tpu_sc as plsc

N = 10000
R = 16
D = 128
NBASES = 10
NLAYERS = 2
NHEADS = 4
E = 320000
BATCH = 1024
P = 20
SEGS = N * R
YR = R + 1
HD = D // NHEADS

NC = 2
NS = 16
NW = NC * NS
EPW = E // NW
KE = 80
NBATCH = EPW // KE

AB = BATCH * (P + 1)
BPW = AB // NW
KG = 96
NGB = BPW // KG

NROWS_BLK = 1000
NBLK = N // NROWS_BLK
BB = 256
SEG_PER_SUB = SEGS // NS
N_PAD = 10240
ROWS_PER_SUB = N_PAD // NS
DRAIN = 32
KW = 25

_mesh = plsc.VectorSubcoreMesh(core_axis_name="c", subcore_axis_name="s")


def _wid():
    return lax.axis_index("s") * NC + lax.axis_index("c")


@functools.partial(
    pl.kernel,
    mesh=_mesh,
    out_type=[
        jax.ShapeDtypeStruct((E,), jnp.int32),
        jax.ShapeDtypeStruct((E,), jnp.int32),
        jax.ShapeDtypeStruct((NC * SEGS,), jnp.float32),
    ],
    scratch_types=[
        pltpu.VMEM((EPW,), jnp.int32),
        pltpu.VMEM((EPW,), jnp.int32),
        pltpu.VMEM((EPW,), jnp.int32),
        pltpu.VMEM((EPW,), jnp.int32),
        pltpu.VMEM((EPW,), jnp.int32),
        pltpu.VMEM((NBATCH, KE), jnp.int32),
        pltpu.VMEM((KE,), jnp.float32),
        pltpu.VMEM((SEG_PER_SUB,), jnp.float32),
        pltpu.VMEM_SHARED((SEGS,), jnp.float32),
        pltpu.SemaphoreType.DMA,
    ],
)
def _sc_prep(src_hbm, dst_hbm, rel_hbm, gidx_hbm, seg_hbm, cntp_hbm,
             src_v, dst_v, rel_v, g_v, s_v, s2_v, ones_v, buf_v, cnt_sh, hsem):
    cid = lax.axis_index("c")
    sid = lax.axis_index("s")
    wid = _wid()
    ebase = wid * EPW

    def zbody(i, _):
        buf_v[pl.ds(i * 16, 16)] = jnp.zeros((16,), jnp.float32)
        return 0

    lax.fori_loop(0, SEG_PER_SUB // 16, zbody, 0)
    for i in range(KE // 16):
        ones_v[pl.ds(i * 16, 16)] = jnp.ones((16,), jnp.float32)
    pltpu.sync_copy(buf_v, cnt_sh.at[pl.ds(sid * SEG_PER_SUB, SEG_PER_SUB)])

    pltpu.sync_copy(src_hbm.at[pl.ds(ebase, EPW)], src_v)
    pltpu.sync_copy(dst_hbm.at[pl.ds(ebase, EPW)], dst_v)
    pltpu.sync_copy(rel_hbm.at[pl.ds(ebase, EPW)], rel_v)

    def vbody(row, _):
        for c in range(KE // 16):
            sl = pl.ds(row * KE + c * 16, 16)
            r = rel_v[sl]
            g_v[sl] = src_v[sl] * YR + r
            seg16 = dst_v[sl] * R + r
            s_v[sl] = seg16
            s2_v[row, pl.ds(c * 16, 16)] = seg16
        return 0

    lax.fori_loop(0, NBATCH, vbody, 0)
    pltpu.sync_copy(g_v, gidx_hbm.at[pl.ds(ebase, EPW)])
    pltpu.sync_copy(s_v, seg_hbm.at[pl.ds(ebase, EPW)])
    plsc.subcore_barrier()

    def hchunk(c, _):
        def hfire(j, _):
            pltpu.async_copy(ones_v, cnt_sh.at[s2_v.at[c * KW + j]], hsem,
                             add=True)
            return 0

        def hwait(j, _):
            pltpu.make_async_copy(ones_v, cnt_sh.at[s2_v.at[c * KW + j]],
                                  hsem).wait()
            return 0

        lax.fori_loop(0, KW, hfire, 0)
        lax.fori_loop(0, KW, hwait, 0)
        return 0

    lax.fori_loop(0, NBATCH // KW, hchunk, 0)
    plsc.subcore_barrier()

    pltpu.sync_copy(cnt_sh.at[pl.ds(sid * SEG_PER_SUB, SEG_PER_SUB)], buf_v)
    pltpu.sync_copy(buf_v, cntp_hbm.at[pl.ds(cid * SEGS + sid * SEG_PER_SUB,
                                             SEG_PER_SUB)])


@functools.partial(
    pl.kernel,
    mesh=_mesh,
    out_type=jax.ShapeDtypeStruct((E,), jnp.float32),
    scratch_types=[
        pltpu.VMEM((EPW,), jnp.int32),
        pltpu.VMEM((EPW,), jnp.int32),
        pltpu.VMEM((NBATCH, KE), jnp.float32),
        pltpu.VMEM((NBATCH, KE), jnp.float32),
        pltpu.SemaphoreType.DMA,
        pltpu.SemaphoreType.DMA,
    ],
)
def _sc_wprep(seg_hbm, cntp_hbm, w_hbm, seg_v, segb_v, c0_v, c1_v, gsem, ssem):
    wid = _wid()
    ebase = wid * EPW
    pltpu.sync_copy(seg_hbm.at[pl.ds(ebase, EPW)], seg_v)

    def obody(i, _):
        sl = pl.ds(i * 16, 16)
        segb_v[sl] = seg_v[sl] + SEGS
        return 0

    lax.fori_loop(0, EPW // 16, obody, 0)

    def chunk(c, _):
        def fire(j, _):
            jj = c * KW + j
            pltpu.async_copy(cntp_hbm.at[seg_v.at[pl.ds(jj * KE, KE)]],
                             c0_v.at[jj], gsem)
            pltpu.async_copy(cntp_hbm.at[segb_v.at[pl.ds(jj * KE, KE)]],
                             c1_v.at[jj], gsem)
            return 0

        def gwait(j, _):
            jj = c * KW + j
            pltpu.make_async_copy(cntp_hbm.at[seg_v.at[pl.ds(jj * KE, KE)]],
                                  c0_v.at[jj], gsem).wait()
            pltpu.make_async_copy(cntp_hbm.at[segb_v.at[pl.ds(jj * KE, KE)]],
                                  c1_v.at[jj], gsem).wait()
            return 0

        def wbody(j, _):
            jj = c * KW + j
            for g in range(KE // 16):
                sl = pl.ds(g * 16, 16)
                c0_v[jj, sl] = 1.0 / jnp.maximum(c0_v[jj, sl] + c1_v[jj, sl], 1.0)
            return 0

        def sfire(j, _):
            jj = c * KW + j
            pltpu.async_copy(c0_v.at[jj], w_hbm.at[pl.ds(ebase + jj * KE, KE)],
                             ssem)
            return 0

        def swait(j, _):
            jj = c * KW + j
            pltpu.make_async_copy(c0_v.at[jj],
                                  w_hbm.at[pl.ds(ebase + jj * KE, KE)],
                                  ssem).wait()
            return 0

        lax.fori_loop(0, KW, fire, 0)
        lax.fori_loop(0, KW, gwait, 0)
        lax.fori_loop(0, KW, wbody, 0)
        lax.fori_loop(0, KW, sfire, 0)
        lax.fori_loop(0, KW, swait, 0)
        return 0

    lax.fori_loop(0, NBATCH // KW, chunk, 0)


@functools.partial(
    pl.kernel,
    mesh=_mesh,
    out_type=jax.ShapeDtypeStruct((NC, N_PAD, D), jnp.float32),
    scratch_types=[
        pltpu.VMEM((EPW,), jnp.int32),
        pltpu.VMEM((EPW,), jnp.float32),
        pltpu.VMEM((2, KE), jnp.int32),
        pltpu.VMEM((2, KE, D), jnp.float32),
        pltpu.VMEM((DRAIN, D), jnp.float32),
        pltpu.VMEM_SHARED((N_PAD, D), jnp.float32),
        pltpu.SemaphoreType.DMA((2,)),
        pltpu.SemaphoreType.DMA((2,)),
        pltpu.SemaphoreType.DMA((2,)),
    ],
)
def _sc_scatter(y_hbm, gidx_hbm, dst_hbm, w_hbm, parts_hbm,
                gidx_v, w_v, dstr_v, rows_v, dbuf, acc_sh,
                dsems, rsems, ssems):
    cid = lax.axis_index("c")
    sid = lax.axis_index("s")
    wid = _wid()
    ebase = wid * EPW

    pltpu.sync_copy(gidx_hbm.at[pl.ds(ebase, EPW)], gidx_v)
    pltpu.sync_copy(w_hbm.at[pl.ds(ebase, EPW)], w_v)

    def zb(i, _):
        for dd in range(D // 16):
            dbuf[i, pl.ds(dd * 16, 16)] = jnp.zeros((16,), jnp.float32)
        return 0

    lax.fori_loop(0, DRAIN, zb, 0)
    for i in range(ROWS_PER_SUB // DRAIN):
        pltpu.sync_copy(dbuf, acc_sh.at[pl.ds(sid * ROWS_PER_SUB + i * DRAIN, DRAIN)])
    plsc.subcore_barrier()

    def fire_batch(j, p):
        pltpu.async_copy(dst_hbm.at[pl.ds(ebase + j * KE, KE)], dstr_v.at[p],
                         dsems.at[p])
        pltpu.async_copy(y_hbm.at[gidx_v.at[pl.ds(j * KE, KE)]], rows_v.at[p],
                         rsems.at[p])

    def wait_batch(j, p):
        pltpu.make_async_copy(dst_hbm.at[pl.ds(ebase + j * KE, KE)],
                              dstr_v.at[p], dsems.at[p]).wait()
        pltpu.make_async_copy(y_hbm.at[gidx_v.at[pl.ds(j * KE, KE)]],
                              rows_v.at[p], rsems.at[p]).wait()

    def wait_scat(p):
        pltpu.make_async_copy(rows_v.at[p], acc_sh.at[dstr_v.at[p]],
                              ssems.at[p]).wait()

    def process(j, p):
        wait_batch(j, p)

        def rb(g2, _):
            wg = w_v[pl.ds(j * KE + g2 * 16, 16)]
            for l in range(16):
                wspl = jnp.full((16,), wg[l], jnp.float32)
                for dd in range(D // 16):
                    sl = pl.ds(dd * 16, 16)
                    rows_v[p, g2 * 16 + l, sl] = rows_v[p, g2 * 16 + l, sl] * wspl
            return 0

        lax.fori_loop(0, KE // 16, rb, 0)
        pltpu.async_copy(rows_v.at[p], acc_sh.at[dstr_v.at[p]], ssems.at[p],
                         add=True)

    fire_batch(0, 0)
    fire_batch(1, 1)

    def group_body(g, _):
        for p in range(2):
            j = 2 * g + p
            process(j, p)

            @pl.when(j + 2 < NBATCH)
            def _():
                wait_scat(p)
                fire_batch(j + 2, p)

        return 0

    lax.fori_loop(0, (NBATCH - 1) // 2, group_body, 0)
    process(NBATCH - 1, 0)
    wait_scat(0)
    wait_scat(1)
    plsc.subcore_barrier()

    for i in range(ROWS_PER_SUB // DRAIN):
        sl = pl.ds(sid * ROWS_PER_SUB + i * DRAIN, DRAIN)
        pltpu.sync_copy(acc_sh.at[sl], dbuf)
        pltpu.sync_copy(dbuf, parts_hbm.at[cid, sl])


@functools.partial(
    pl.kernel,
    mesh=_mesh,
    out_type=jax.ShapeDtypeStruct((AB, D), jnp.float32),
    scratch_types=[
        pltpu.VMEM((KG,), jnp.int32),
        pltpu.VMEM((KG, D), jnp.float32),
    ],
)
def _sc_gather(x_hbm, idx_hbm, out_hbm, idx_v, rows_v):
    wid = _wid()

    def gb(i, _):
        base = wid * BPW + i * KG
        pltpu.sync_copy(idx_hbm.at[pl.ds(base, KG)], idx_v)
        pltpu.sync_copy(x_hbm.at[idx_v], rows_v)
        pltpu.sync_copy(rows_v, out_hbm.at[pl.ds(base, KG)])
        return 0

    lax.fori_loop(0, NGB, gb, 0)


def _y_body(x_ref, basis_ref, comp_ref, root_ref, y_ref):
    x = x_ref[...]
    comp = comp_ref[...]
    xbs = [x @ basis_ref[b] for b in range(NBASES)]
    cols = []
    for r in range(R):
        acc = comp[r, 0] * xbs[0]
        for b in range(1, NBASES):
            acc = acc + comp[r, b] * xbs[b]
        cols.append(acc)
    cols.append(x @ root_ref[...])
    y_ref[...] = jnp.concatenate(cols, axis=1)


def _tc_y(x, basis_l, comp_l, root_l):
    return pl.pallas_call(
        _y_body,
        grid=(NBLK,),
        in_specs=[
            pl.BlockSpec((NROWS_BLK, D), lambda i: (i, 0)),
            pl.BlockSpec((NBASES, D, D), lambda i: (0, 0, 0)),
            pl.BlockSpec((R, NBASES), lambda i: (0, 0)),
            pl.BlockSpec((D, D), lambda i: (0, 0)),
        ],
        out_specs=pl.BlockSpec((NROWS_BLK, YR * D), lambda i: (i, 0)),
        out_shape=jax.ShapeDtypeStruct((N, YR * D), jnp.float32),
    )(x, basis_l, comp_l, root_l)


def _comb_body(parts_ref, yroot_ref, x_ref, cb_ref, ls_ref, lb_ref, o_ref):
    h = parts_ref[0] + parts_ref[1] + yroot_ref[...] + cb_ref[...]
    mu = jnp.mean(h, axis=-1, keepdims=True)
    var = jnp.mean((h - mu) ** 2, axis=-1, keepdims=True)
    hn = (h - mu) / jnp.sqrt(var + 1e-5) * ls_ref[...] + lb_ref[...]
    o_ref[...] = jnp.maximum(hn, 0.0) + x_ref[...]


def _tc_combine(parts, y, x, cb, ls, lb):
    return pl.pallas_call(
        _comb_body,
        grid=(NBLK,),
        in_specs=[
            pl.BlockSpec((NC, NROWS_BLK, D), lambda i: (0, i, 0)),
            pl.BlockSpec((NROWS_BLK, D), lambda i: (i, R)),
            pl.BlockSpec((NROWS_BLK, D), lambda i: (i, 0)),
            pl.BlockSpec((1, D), lambda i: (0, 0)),
            pl.BlockSpec((1, D), lambda i: (0, 0)),
            pl.BlockSpec((1, D), lambda i: (0, 0)),
        ],
        out_specs=pl.BlockSpec((NROWS_BLK, D), lambda i: (i, 0)),
        out_shape=jax.ShapeDtypeStruct((N, D), jnp.float32),
    )(parts, y, x, cb, ls, lb)


def _attn_body(drug_ref, ph_ref, mask_ref, wq_ref, wk_ref, wv_ref, wo_ref,
               bq_ref, bk_ref, bv_ref, bo_ref, sw_ref, sb_ref, out_ref):
    drug = drug_ref[...]
    ph = ph_ref[...]
    mask = mask_ref[...]
    q = drug @ wq_ref[...] + bq_ref[...]
    phf = ph.reshape(BB * P, D)
    k = (phf @ wk_ref[...] + bk_ref[...]).reshape(BB, P, D)
    v = (phf @ wv_ref[...] + bv_ref[...]).reshape(BB, P, D)
    scale = 1.0 / jnp.sqrt(jnp.float32(HD))
    ctxs = []
    for h in range(NHEADS):
        qh = q[:, h * HD:(h + 1) * HD]
        kh = k[:, :, h * HD:(h + 1) * HD]
        vh = v[:, :, h * HD:(h + 1) * HD]
        logit = jnp.sum(qh[:, None, :] * kh, axis=-1) * scale
        logit = jnp.where(mask > 0.0, jnp.float32(-1e9), logit)
        m = jnp.max(logit, axis=-1, keepdims=True)
        e = jnp.exp(logit - m)
        a = e / jnp.sum(e, axis=-1, keepdims=True)
        ctxs.append(jnp.sum(a[:, :, None] * vh, axis=1))
    ctx = jnp.concatenate(ctxs, axis=-1)
    out = ctx @ wo_ref[...] + bo_ref[...]
    score = jnp.sum(out * sw_ref[...], axis=-1) + sb_ref[0, 0]
    out_ref[...] = score[:, None]


def _tc_attn(drug, ph, maskf, Wq, Wk, Wv, Wo, bq, bk, bv, bo, sw, sb):
    full = lambda *shape: pl.BlockSpec(shape, lambda i: tuple(0 for _ in shape))
    return pl.pallas_call(
        _attn_body,
        grid=(BATCH // BB,),
        in_specs=[
            pl.BlockSpec((BB, D), lambda i: (i, 0)),
            pl.BlockSpec((BB, P, D), lambda i: (i, 0, 0)),
            pl.BlockSpec((BB, P), lambda i: (i, 0)),
            full(D, D), full(D, D), full(D, D), full(D, D),
            full(1, D), full(1, D), full(1, D), full(1, D),
            full(1, D), full(1, 1),
        ],
        out_specs=pl.BlockSpec((BB, 1), lambda i: (i, 0)),
        out_shape=jax.ShapeDtypeStruct((BATCH, 1), jnp.float32),
    )(drug, ph, maskf, Wq, Wk, Wv, Wo, bq, bk, bv, bo, sw, sb)


def kernel(node_emb, basis, comp, root_w, conv_bias, ln_scale, ln_bias,
           Wq, Wk, Wv, Wo, bq, bk, bv, bo, score_w, score_b,
           edge_index, edge_type, drug_indices, pheno_indices, pheno_mask):
    src = edge_index[0]
    dst = edge_index[1]
    gidx, seg, cntp = _sc_prep(src, dst, edge_type)
    w = _sc_wprep(seg, cntp)

    x = node_emb
    for l in range(NLAYERS):
        y = _tc_y(x, basis[l], comp[l], root_w[l])
        y2 = y.reshape(N * YR, D)
        parts = _sc_scatter(y2, gidx, dst, w)
        x = _tc_combine(parts, y, x,
                        conv_bias[l].reshape(1, D),
                        ln_scale[l].reshape(1, D),
                        ln_bias[l].reshape(1, D))

    all_idx = jnp.concatenate([drug_indices, pheno_indices.reshape(-1)], axis=0)
    rows = _sc_gather(x, all_idx)
    drug = rows[:BATCH]
    ph = rows[BATCH:].reshape(BATCH, P, D)
    score = _tc_attn(drug, ph, pheno_mask.astype(jnp.float32),
                     Wq, Wk, Wv, Wo,
                     bq.reshape(1, D), bk.reshape(1, D), bv.reshape(1, D),
                     bo.reshape(1, D), score_w.reshape(1, D),
                     score_b.reshape(1, 1))
    return score.reshape(BATCH)
